# Initial kernel scaffold; baseline (speedup 1.0000x reference)
import jax
import jax.numpy as jnp
from jax.experimental import pallas as pl


def kernel(x, edge_index, edge_type, goalVec, goalObjectsVec, params):
    raise NotImplementedError("write your pallas kernel here")


# speedup vs baseline: 12.3999x; 12.3999x over previous
import functools

import jax
import jax.numpy as jnp
from jax import lax
from jax.experimental import pallas as pl
from jax.experimental.pallas import tpu as pltpu
from jax.experimental.pallas import tpu_sc as plsc

N_NODES = 10000
N_HIDDEN = 64
N_ETYPES = 4

NC = 2
NS = 16
NW = NC * NS
CHUNK = 128
ACC_ROWS = 10112
RPS = ACC_ROWS // NS
DUMMY_DST = N_NODES


@functools.cache
def _make_sc_agg(cpw: int, table_rows: int):
    mesh = plsc.VectorSubcoreMesh(core_axis_name="c", subcore_axis_name="s")

    @functools.partial(
        pl.kernel,
        mesh=mesh,
        compiler_params=pltpu.CompilerParams(use_tc_tiling_on_sc=False),
        out_type=jax.ShapeDtypeStruct((NC, ACC_ROWS, N_HIDDEN), jnp.float32),
        scratch_types=[
            pltpu.VMEM((CHUNK,), jnp.int32),
            pltpu.VMEM((CHUNK,), jnp.int32),
            pltpu.VMEM((CHUNK, N_HIDDEN), jnp.float32),
            pltpu.VMEM((RPS, N_HIDDEN), jnp.float32),
            pltpu.VMEM_SHARED((ACC_ROWS, N_HIDDEN), jnp.float32),
            pltpu.SemaphoreType.DMA,
        ],
    )
    def sc_agg(rows_hbm, dst_hbm, table_hbm, out_hbm,
               ridx, didx, gbuf, zbuf, acc, sem):
        c = lax.axis_index("c")
        s = lax.axis_index("s")
        w = c * NS + s

        def _zrow(i, carry):
            for c4 in range(N_HIDDEN // 16):
                zbuf[i, pl.ds(c4 * 16, 16)] = jnp.zeros((16,), jnp.float32)
            return carry
        lax.fori_loop(0, RPS, _zrow, 0)
        pltpu.sync_copy(zbuf, acc.at[pl.ds(s * RPS, RPS)])
        plsc.subcore_barrier()

        def _chunk(j, carry):
            pltpu.sync_copy(rows_hbm.at[w, j], ridx)
            pltpu.sync_copy(dst_hbm.at[w, j], didx)
            pltpu.async_copy(table_hbm.at[ridx], gbuf, sem).wait()
            pltpu.sync_copy(gbuf, acc.at[didx], add=True)
            return carry
        lax.fori_loop(0, cpw, _chunk, 0)
        plsc.subcore_barrier()

        pltpu.sync_copy(acc.at[pl.ds(s * RPS, RPS)],
                        out_hbm.at[c, pl.ds(s * RPS, RPS)])

    return sc_agg


_BN = 1000


@functools.cache
def _make_pre(di: int):
    def body(h_ref, w_ref, b_ref, o_ref):
        r = pl.program_id(1)
        o_ref[0] = (jnp.dot(h_ref[...], w_ref[0],
                            preferred_element_type=jnp.float32)
                    + b_ref[r][None, :])

    return pl.pallas_call(
        body,
        grid=(N_NODES // _BN, N_ETYPES),
        in_specs=[
            pl.BlockSpec((_BN, di), lambda n, r: (n, 0)),
            pl.BlockSpec((1, di, N_HIDDEN), lambda n, r: (r, 0, 0)),
            pl.BlockSpec((N_ETYPES, N_HIDDEN), lambda n, r: (0, 0)),
        ],
        out_specs=pl.BlockSpec((1, _BN, N_HIDDEN), lambda n, r: (r, n, 0)),
        out_shape=jax.ShapeDtypeStruct((N_ETYPES, N_NODES, N_HIDDEN),
                                       jnp.float32),
    )


@functools.cache
def _make_post(di: int):
    def body(h_ref, p_ref, wg_ref, ug_ref, ws_ref, bg_ref, o_ref):
        hb = h_ref[...]
        agg = p_ref[0] + p_ref[1]
        z = (jnp.dot(hb, wg_ref[...], preferred_element_type=jnp.float32)
             + jnp.dot(agg, ug_ref[...], preferred_element_type=jnp.float32)
             + bg_ref[...][None, :])
        gate = jax.nn.sigmoid(z)
        self_t = jnp.dot(hb, ws_ref[...], preferred_element_type=jnp.float32)
        o_ref[...] = jnp.maximum(gate * agg + (1.0 - gate) * self_t, 0.0)

    return pl.pallas_call(
        body,
        grid=(N_NODES // _BN,),
        in_specs=[
            pl.BlockSpec((_BN, di), lambda n: (n, 0)),
            pl.BlockSpec((NC, _BN, N_HIDDEN), lambda n: (0, n, 0)),
            pl.BlockSpec((di, N_HIDDEN), lambda n: (0, 0)),
            pl.BlockSpec((N_HIDDEN, N_HIDDEN), lambda n: (0, 0)),
            pl.BlockSpec((di, N_HIDDEN), lambda n: (0, 0)),
            pl.BlockSpec((N_HIDDEN,), lambda n: (0,)),
        ],
        out_specs=pl.BlockSpec((_BN, N_HIDDEN), lambda n: (n, 0)),
        out_shape=jax.ShapeDtypeStruct((N_NODES, N_HIDDEN), jnp.float32),
    )


_KB = 25624
_KSTEPS = 25


def _fc1_body(hc_ref, w_ref, o_ref):
    @pl.when(pl.program_id(0) == 0)
    def _():
        o_ref[...] = jnp.zeros_like(o_ref)

    o_ref[...] += lax.dot_general(
        hc_ref[...], w_ref[...], (((0,), (0,)), ((), ())),
        preferred_element_type=jnp.float32)


def _make_fc1(k_total: int):
    assert k_total == _KB * _KSTEPS
    return pl.pallas_call(
        _fc1_body,
        grid=(_KSTEPS,),
        in_specs=[
            pl.BlockSpec((_KB, 1), lambda k: (k, 0)),
            pl.BlockSpec((_KB, N_HIDDEN), lambda k: (k, 0)),
        ],
        out_specs=pl.BlockSpec((1, N_HIDDEN), lambda k: (0, 0)),
        out_shape=jax.ShapeDtypeStruct((1, N_HIDDEN), jnp.float32),
    )


def _head_body(f_ref, b1_ref, w2_ref, b2_ref, w3_ref, b3_ref, o_ref):
    h1 = jnp.maximum(f_ref[...] + b1_ref[...][None, :], 0.0)
    h2 = jnp.maximum(
        jnp.dot(h1, w2_ref[...], preferred_element_type=jnp.float32)
        + b2_ref[...][None, :], 0.0)
    o_ref[...] = jax.nn.sigmoid(
        jnp.dot(h2, w3_ref[...], preferred_element_type=jnp.float32)
        + b3_ref[...][None, :])


def _make_head(n_classes: int):
    return pl.pallas_call(
        _head_body,
        out_shape=jax.ShapeDtypeStruct((1, n_classes), jnp.float32),
    )


def kernel(x, edge_index, edge_type, goalVec, goalObjectsVec, params):
    src = edge_index[0].astype(jnp.int32)
    dst = edge_index[1].astype(jnp.int32)
    et = edge_type.astype(jnp.int32)
    e = src.shape[0]
    cpw = -(-e // (NW * CHUNK))
    e_pad = NW * cpw * CHUNK

    rows = et * N_NODES + src
    rows = jnp.concatenate(
        [rows, jnp.zeros((e_pad - e,), jnp.int32)]).reshape(NW, cpw, CHUNK)
    dstp = jnp.concatenate(
        [dst, jnp.full((e_pad - e,), DUMMY_DST, jnp.int32)]
    ).reshape(NW, cpw, CHUNK)

    sc_agg = _make_sc_agg(cpw, N_ETYPES * N_NODES)

    h = x
    for p in params['layers']:
        di = h.shape[1]
        table = _make_pre(di)(h, p['W_rel'], p['b_rel'])
        table = table.reshape(N_ETYPES * N_NODES, N_HIDDEN)
        partials = sc_agg(rows, dstp, table)
        h = _make_post(di)(h, partials, p['W_gate'], p['U_gate'],
                           p['W_self'], p['b_gate'])

    hcat = jnp.concatenate([h.reshape(-1), goalVec, goalObjectsVec])
    f1 = _make_fc1(hcat.shape[0])(hcat.reshape(-1, 1), params['fc1_w'])
    out = _make_head(params['fc3_w'].shape[1])(
        f1, params['fc1_b'], params['fc2_w'], params['fc2_b'],
        params['fc3_w'], params['fc3_b'])
    return out.reshape(-1)


# speedup vs baseline: 12.9527x; 1.0446x over previous
import functools

import jax
import jax.numpy as jnp
from jax import lax
from jax.experimental import pallas as pl
from jax.experimental.pallas import tpu as pltpu
from jax.experimental.pallas import tpu_sc as plsc

N_NODES = 10000
N_HIDDEN = 64
N_ETYPES = 4

NC = 2
NS = 16
NW = NC * NS
CHUNK = 128
ACC_ROWS = 10112
RPS = ACC_ROWS // NS
DUMMY_DST = N_NODES


@functools.cache
def _make_sc_agg(cpw: int, table_rows: int):
    assert cpw % 2 == 0
    mesh = plsc.VectorSubcoreMesh(core_axis_name="c", subcore_axis_name="s")

    @functools.partial(
        pl.kernel,
        mesh=mesh,
        compiler_params=pltpu.CompilerParams(use_tc_tiling_on_sc=False),
        out_type=jax.ShapeDtypeStruct((NC, ACC_ROWS, N_HIDDEN), jnp.float32),
        scratch_types=[
            pltpu.VMEM((cpw, CHUNK), jnp.int32),
            pltpu.VMEM((cpw, CHUNK), jnp.int32),
            pltpu.VMEM((CHUNK, N_HIDDEN), jnp.float32),
            pltpu.VMEM((CHUNK, N_HIDDEN), jnp.float32),
            pltpu.VMEM((RPS, N_HIDDEN), jnp.float32),
            pltpu.VMEM_SHARED((ACC_ROWS, N_HIDDEN), jnp.float32),
            pltpu.SemaphoreType.DMA,
            pltpu.SemaphoreType.DMA,
        ],
    )
    def sc_agg(rows_hbm, dst_hbm, table_hbm, out_hbm,
               rowbuf, dstbuf, gbuf0, gbuf1, zbuf, acc, sem0, sem1):
        c = lax.axis_index("c")
        s = lax.axis_index("s")
        w = c * NS + s

        cp_r = pltpu.async_copy(rows_hbm.at[w], rowbuf, sem0)
        cp_d = pltpu.async_copy(dst_hbm.at[w], dstbuf, sem1)

        def _zrow(i, carry):
            for c4 in range(N_HIDDEN // 16):
                zbuf[i, pl.ds(c4 * 16, 16)] = jnp.zeros((16,), jnp.float32)
            return carry
        lax.fori_loop(0, RPS, _zrow, 0)
        pltpu.sync_copy(zbuf, acc.at[pl.ds(s * RPS, RPS)])
        cp_r.wait()
        cp_d.wait()
        plsc.subcore_barrier()

        nsteps = cpw // 2
        pltpu.async_copy(table_hbm.at[rowbuf.at[0]], gbuf0, sem0)

        def _step(t, carry):
            j = 2 * t
            pltpu.async_copy(table_hbm.at[rowbuf.at[j + 1]], gbuf1, sem1)
            pltpu.make_async_copy(table_hbm.at[rowbuf.at[j]], gbuf0,
                                  sem0).wait()
            pltpu.sync_copy(gbuf0, acc.at[dstbuf.at[j]], add=True)

            @pl.when(t + 1 < nsteps)
            def _():
                pltpu.async_copy(table_hbm.at[rowbuf.at[j + 2]], gbuf0, sem0)

            pltpu.make_async_copy(table_hbm.at[rowbuf.at[j + 1]], gbuf1,
                                  sem1).wait()
            pltpu.sync_copy(gbuf1, acc.at[dstbuf.at[j + 1]], add=True)
            return carry

        lax.fori_loop(0, nsteps, _step, 0)
        plsc.subcore_barrier()

        pltpu.sync_copy(acc.at[pl.ds(s * RPS, RPS)],
                        out_hbm.at[c, pl.ds(s * RPS, RPS)])

    return sc_agg


_BN = 1000


@functools.cache
def _make_pre(di: int):
    def body(h_ref, w_ref, b_ref, o_ref):
        r = pl.program_id(1)
        o_ref[0] = (jnp.dot(h_ref[...], w_ref[0],
                            preferred_element_type=jnp.float32)
                    + b_ref[r][None, :])

    return pl.pallas_call(
        body,
        grid=(N_NODES // _BN, N_ETYPES),
        in_specs=[
            pl.BlockSpec((_BN, di), lambda n, r: (n, 0)),
            pl.BlockSpec((1, di, N_HIDDEN), lambda n, r: (r, 0, 0)),
            pl.BlockSpec((N_ETYPES, N_HIDDEN), lambda n, r: (0, 0)),
        ],
        out_specs=pl.BlockSpec((1, _BN, N_HIDDEN), lambda n, r: (r, n, 0)),
        out_shape=jax.ShapeDtypeStruct((N_ETYPES, N_NODES, N_HIDDEN),
                                       jnp.float32),
    )


@functools.cache
def _make_post(di: int):
    def body(h_ref, p_ref, wg_ref, ug_ref, ws_ref, bg_ref, o_ref):
        hb = h_ref[...]
        agg = p_ref[0] + p_ref[1]
        z = (jnp.dot(hb, wg_ref[...], preferred_element_type=jnp.float32)
             + jnp.dot(agg, ug_ref[...], preferred_element_type=jnp.float32)
             + bg_ref[...][None, :])
        gate = jax.nn.sigmoid(z)
        self_t = jnp.dot(hb, ws_ref[...], preferred_element_type=jnp.float32)
        o_ref[...] = jnp.maximum(gate * agg + (1.0 - gate) * self_t, 0.0)

    return pl.pallas_call(
        body,
        grid=(N_NODES // _BN,),
        in_specs=[
            pl.BlockSpec((_BN, di), lambda n: (n, 0)),
            pl.BlockSpec((NC, _BN, N_HIDDEN), lambda n: (0, n, 0)),
            pl.BlockSpec((di, N_HIDDEN), lambda n: (0, 0)),
            pl.BlockSpec((N_HIDDEN, N_HIDDEN), lambda n: (0, 0)),
            pl.BlockSpec((di, N_HIDDEN), lambda n: (0, 0)),
            pl.BlockSpec((N_HIDDEN,), lambda n: (0,)),
        ],
        out_specs=pl.BlockSpec((_BN, N_HIDDEN), lambda n: (n, 0)),
        out_shape=jax.ShapeDtypeStruct((N_NODES, N_HIDDEN), jnp.float32),
    )


_KB = 25624
_KSTEPS = 25


def _fc1_body(hc_ref, w_ref, o_ref):
    @pl.when(pl.program_id(0) == 0)
    def _():
        o_ref[...] = jnp.zeros_like(o_ref)

    o_ref[...] += lax.dot_general(
        hc_ref[...], w_ref[...], (((0,), (0,)), ((), ())),
        preferred_element_type=jnp.float32)


def _make_fc1(k_total: int):
    assert k_total == _KB * _KSTEPS
    return pl.pallas_call(
        _fc1_body,
        grid=(_KSTEPS,),
        in_specs=[
            pl.BlockSpec((_KB, 1), lambda k: (k, 0)),
            pl.BlockSpec((_KB, N_HIDDEN), lambda k: (k, 0)),
        ],
        out_specs=pl.BlockSpec((1, N_HIDDEN), lambda k: (0, 0)),
        out_shape=jax.ShapeDtypeStruct((1, N_HIDDEN), jnp.float32),
    )


def _head_body(f_ref, b1_ref, w2_ref, b2_ref, w3_ref, b3_ref, o_ref):
    h1 = jnp.maximum(f_ref[...] + b1_ref[...][None, :], 0.0)
    h2 = jnp.maximum(
        jnp.dot(h1, w2_ref[...], preferred_element_type=jnp.float32)
        + b2_ref[...][None, :], 0.0)
    o_ref[...] = jax.nn.sigmoid(
        jnp.dot(h2, w3_ref[...], preferred_element_type=jnp.float32)
        + b3_ref[...][None, :])


def _make_head(n_classes: int):
    return pl.pallas_call(
        _head_body,
        out_shape=jax.ShapeDtypeStruct((1, n_classes), jnp.float32),
    )


def kernel(x, edge_index, edge_type, goalVec, goalObjectsVec, params):
    src = edge_index[0].astype(jnp.int32)
    dst = edge_index[1].astype(jnp.int32)
    et = edge_type.astype(jnp.int32)
    e = src.shape[0]
    cpw = -(-e // (NW * CHUNK))
    cpw += cpw % 2
    e_pad = NW * cpw * CHUNK

    rows = et * N_NODES + src
    rows = jnp.concatenate(
        [rows, jnp.zeros((e_pad - e,), jnp.int32)]).reshape(NW, cpw, CHUNK)
    dstp = jnp.concatenate(
        [dst, jnp.full((e_pad - e,), DUMMY_DST, jnp.int32)]
    ).reshape(NW, cpw, CHUNK)

    sc_agg = _make_sc_agg(cpw, N_ETYPES * N_NODES)

    h = x
    for p in params['layers']:
        di = h.shape[1]
        table = _make_pre(di)(h, p['W_rel'], p['b_rel'])
        table = table.reshape(N_ETYPES * N_NODES, N_HIDDEN)
        partials = sc_agg(rows, dstp, table)
        h = _make_post(di)(h, partials, p['W_gate'], p['U_gate'],
                           p['W_self'], p['b_gate'])

    hcat = jnp.concatenate([h.reshape(-1), goalVec, goalObjectsVec])
    f1 = _make_fc1(hcat.shape[0])(hcat.reshape(-1, 1), params['fc1_w'])
    out = _make_head(params['fc3_w'].shape[1])(
        f1, params['fc1_b'], params['fc2_w'], params['fc2_b'],
        params['fc3_w'], params['fc3_b'])
    return out.reshape(-1)
